# Initial kernel scaffold; baseline (speedup 1.0000x reference)
import jax
import jax.numpy as jnp
from jax.experimental import pallas as pl


def kernel(x, edge_index, W1, b1, W2, b2, W3, b3):
    raise NotImplementedError("write your pallas kernel here")


# speedup vs baseline: 4.9443x; 4.9443x over previous
import functools

import jax
import jax.numpy as jnp
from jax import lax
from jax.experimental import pallas as pl
from jax.experimental.pallas import tpu as pltpu
from jax.experimental.pallas import tpu_sc as plsc

N = 10000
E = 320000
D = 128

NC = 2
NS = 16
NW = NC * NS
EDGES_PER_WORKER = E // NW
BLK = 80
NBLK = EDGES_PER_WORKER // BLK
ROWS_PER_SUB = N // NS


def _sc_aggregate(values, zeros, src, dst):
    mesh = plsc.VectorSubcoreMesh(core_axis_name="c", subcore_axis_name="s")

    @functools.partial(
        pl.kernel,
        out_type=jax.ShapeDtypeStruct((NC, N, D), jnp.float32),
        mesh=mesh,
        scratch_types=[
            pltpu.VMEM((BLK,), jnp.int32),
            pltpu.VMEM((BLK,), jnp.int32),
            pltpu.VMEM((BLK, D), jnp.float32),
            pltpu.VMEM_SHARED((N, D), jnp.float32),
        ],
    )
    def agg_kernel(x_hbm, z_hbm, src_hbm, dst_hbm, out_hbm,
                   src_v, dst_v, rows_v, acc_sh):
        cid = lax.axis_index("c")
        sid = lax.axis_index("s")
        wid = sid * NC + cid

        @pl.when(sid == 0)
        def _():
            pltpu.sync_copy(z_hbm, acc_sh)

        plsc.subcore_barrier()

        @pl.loop(0, NBLK)
        def _(j):
            base = wid * EDGES_PER_WORKER + j * BLK
            pltpu.sync_copy(src_hbm.at[pl.ds(base, BLK)], src_v)
            pltpu.sync_copy(dst_hbm.at[pl.ds(base, BLK)], dst_v)
            pltpu.sync_copy(x_hbm.at[src_v], rows_v)
            pltpu.sync_copy(rows_v, acc_sh.at[dst_v], add=True)

        plsc.subcore_barrier()

        @pl.when(sid == 0)
        def _():
            pltpu.sync_copy(acc_sh, out_hbm.at[cid])

    return agg_kernel(values, zeros, src, dst)


ROW_BLK = 1000


def _mlp1_body(x_ref, p_ref, w1_ref, b1_ref, w2_ref, b2_ref, o_ref):
    h = x_ref[...] + p_ref[0] + p_ref[1]
    a = lax.dot_general(h, w1_ref[...], (((1,), (0,)), ((), ())),
                        precision=lax.Precision.HIGHEST,
                        preferred_element_type=jnp.float32)
    a = jnp.maximum(a + b1_ref[...], 0.0)
    hh = lax.dot_general(a, w2_ref[...], (((1,), (0,)), ((), ())),
                         precision=lax.Precision.HIGHEST,
                         preferred_element_type=jnp.float32)
    hh = hh + b2_ref[...]
    o_ref[...] = jnp.where(hh > 0, hh, jnp.exp(hh) - 1.0)


def _mlp2_body(h_ref, q_ref, w3_ref, b3_ref, o_ref):
    h2 = h_ref[...] + q_ref[0] + q_ref[1]
    a = lax.dot_general(h2, w3_ref[...], (((1,), (0,)), ((), ())),
                        precision=lax.Precision.HIGHEST,
                        preferred_element_type=jnp.float32)
    o_ref[...] = jnp.maximum(a + b3_ref[...], 0.0)


def _row_spec():
    return pl.BlockSpec((ROW_BLK, D), lambda i: (i, 0))


def _pair_spec():
    return pl.BlockSpec((NC, ROW_BLK, D), lambda i: (0, i, 0))


def _full_spec(shape):
    return pl.BlockSpec(shape, lambda i: tuple(0 for _ in shape))


def _mlp1(x, p, W1, b1, W2, b2):
    return pl.pallas_call(
        _mlp1_body,
        grid=(N // ROW_BLK,),
        in_specs=[_row_spec(), _pair_spec(),
                  _full_spec((D, D)), _full_spec((1, D)),
                  _full_spec((D, D)), _full_spec((1, D))],
        out_specs=_row_spec(),
        out_shape=jax.ShapeDtypeStruct((N, D), jnp.float32),
    )(x, p, W1, b1.reshape(1, D), W2, b2.reshape(1, D))


def _mlp2(h, q, W3, b3):
    return pl.pallas_call(
        _mlp2_body,
        grid=(N // ROW_BLK,),
        in_specs=[_row_spec(), _pair_spec(),
                  _full_spec((D, D)), _full_spec((1, D))],
        out_specs=_row_spec(),
        out_shape=jax.ShapeDtypeStruct((N, D), jnp.float32),
    )(h, q, W3, b3.reshape(1, D))


def kernel(x, edge_index, W1, b1, W2, b2, W3, b3):
    src = edge_index[0].astype(jnp.int32)
    dst = edge_index[1].astype(jnp.int32)
    zeros = jnp.zeros((N, D), jnp.float32)

    p = _sc_aggregate(x, zeros, src, dst)
    h = _mlp1(x, p, W1, b1, W2, b2)
    q = _sc_aggregate(h, zeros, src, dst)
    return _mlp2(h, q, W3, b3)
